# baseline (speedup 1.0000x reference)
import functools

import jax
import jax.numpy as jnp
from jax import lax
from jax.experimental import pallas as pl
from jax.experimental.pallas import tpu as pltpu
from jax.experimental.pallas import tpu_sc as plsc

P = 16
Q = 32
S = 2048
D = 256
TOTAL_Q = P * Q

L = 16
NW = 32
QB = TOTAL_Q // NW

SCHUNK = 512
NSBLK = S // SCHUNK


def _costs_body(problems_ref, valid_ref, w_ref, costs_ref):
    c = jnp.dot(problems_ref[...], w_ref[...], preferred_element_type=jnp.float32)
    vf = valid_ref[...].astype(jnp.float32)
    costs_ref[...] = c * vf.reshape(P, 1)


def _sc_reduce_body(costs_hbm, q_hbm, out_hbm, costs_v, q_v, out_v, sem):
    wid = lax.axis_index("s") * 2 + lax.axis_index("c")
    qbase = wid * QB
    prob = wid // 2

    cp = pltpu.async_copy(q_hbm.at[pl.ds(qbase * S, QB * S)], q_v, sem)
    pltpu.sync_copy(costs_hbm.at[prob], costs_v)
    cp.wait()

    def body(c, accs):
        cc = costs_v[pl.ds(c * L, L)]
        return tuple(accs[i] + q_v[pl.ds(i * S + c * L, L)] * cc
                     for i in range(QB))

    zero = jnp.zeros((L,), jnp.float32)
    accs = lax.fori_loop(0, S // L, body, tuple(zero for _ in range(QB)))

    lanes = lax.iota(jnp.int32, L)
    tot = zero
    for i in range(QB):
        tot = jnp.where(lanes == i, jnp.sum(accs[i]), tot)
    out_v[...] = tot
    pltpu.sync_copy(out_v, out_hbm.at[pl.ds(qbase, QB)])


_sc_reduce = functools.partial(
    pl.kernel,
    out_type=jax.ShapeDtypeStruct((TOTAL_Q,), jnp.float32),
    mesh=plsc.VectorSubcoreMesh(core_axis_name="c", subcore_axis_name="s"),
    compiler_params=pltpu.CompilerParams(needs_layout_passes=False),
    scratch_types=[
        pltpu.VMEM((S,), jnp.float32),
        pltpu.VMEM((QB * S,), jnp.float32),
        pltpu.VMEM((L,), jnp.float32),
        pltpu.SemaphoreType.DMA,
    ],
)(_sc_reduce_body)


def kernel(problems, questions_flat_values, questions_outer_row_splits,
           questions_inner_row_splits, valid, W):
    costs = pl.pallas_call(
        _costs_body,
        grid=(NSBLK,),
        in_specs=[
            pl.BlockSpec((P, D), lambda i: (0, 0)),
            pl.BlockSpec((P,), lambda i: (0,)),
            pl.BlockSpec((D, SCHUNK), lambda i: (0, i)),
        ],
        out_specs=pl.BlockSpec((P, SCHUNK), lambda i: (0, i)),
        out_shape=jax.ShapeDtypeStruct((P, S), jnp.float32),
    )(problems, valid, W)
    return _sc_reduce(costs, questions_flat_values)
